# baseline (speedup 1.0000x reference)
import dataclasses
import functools

import jax
import jax.numpy as jnp
from jax import lax
from jax.experimental import pallas as pl
from jax.experimental.pallas import tpu as pltpu
from jax.experimental.pallas import tpu_sc as plsc

LATENT = 32
LANES = 16
NUM_CORES = 2
NUM_SUBCORES = 16
NUM_WORKERS = NUM_CORES * NUM_SUBCORES
GATHER_CHUNK = 128


def _scores_sc(uidx, iidx, user_table, item_table, w_flat, bias16):
    batch = uidx.shape[0]
    b_per_w = batch // NUM_WORKERS
    n_chunks = b_per_w // GATHER_CHUNK
    n_groups = b_per_w // LANES

    mesh = plsc.VectorSubcoreMesh(core_axis_name="c", subcore_axis_name="s")
    cparams = pltpu.CompilerParams()
    if "needs_layout_passes" in pltpu.CompilerParams.__dataclass_fields__:
        cparams = dataclasses.replace(cparams, needs_layout_passes=False)
    if "use_tc_tiling_on_sc" in pltpu.CompilerParams.__dataclass_fields__:
        cparams = dataclasses.replace(cparams, use_tc_tiling_on_sc=False)

    @functools.partial(
        pl.kernel,
        mesh=mesh,
        compiler_params=cparams,
        out_type=jax.ShapeDtypeStruct((batch,), jnp.float32),
        scratch_types=[
            pltpu.VMEM((b_per_w,), jnp.int32),
            pltpu.VMEM((b_per_w,), jnp.int32),
            pltpu.VMEM((b_per_w, LATENT), jnp.float32),
            pltpu.VMEM((b_per_w, LATENT), jnp.float32),
            pltpu.VMEM((LANES * LANES,), jnp.float32),
            pltpu.VMEM((b_per_w,), jnp.float32),
            pltpu.VMEM((LATENT,), jnp.float32),
            pltpu.VMEM((LANES,), jnp.float32),
            pltpu.SemaphoreType.DMA,
            pltpu.SemaphoreType.DMA,
        ],
    )
    def k(uidx_hbm, iidx_hbm, utab_hbm, itab_hbm, w_hbm, b_hbm, out_hbm,
          uidx_v, iidx_v, urows_v, irows_v, q_v, scores_v, w_v, b_v,
          sem_u, sem_i):
        wid = lax.axis_index("s") * NUM_CORES + lax.axis_index("c")
        base = wid * b_per_w

        pltpu.sync_copy(uidx_hbm.at[pl.ds(base, b_per_w)], uidx_v)
        pltpu.sync_copy(iidx_hbm.at[pl.ds(base, b_per_w)], iidx_v)
        pltpu.sync_copy(w_hbm, w_v)
        pltpu.sync_copy(b_hbm, b_v)

        copies = []
        for c in range(n_chunks):
            sl = pl.ds(c * GATHER_CHUNK, GATHER_CHUNK)
            copies.append(pltpu.async_copy(
                utab_hbm.at[uidx_v.at[sl]], urows_v.at[sl], sem_u))
            copies.append(pltpu.async_copy(
                itab_hbm.at[iidx_v.at[sl]], irows_v.at[sl], sem_i))
        for cp in copies:
            cp.wait()

        w_lo = w_v[pl.ds(0, LANES)]
        w_hi = w_v[pl.ds(LANES, LANES)]
        bvec = b_v[...]
        lane_off = lax.iota(jnp.int32, LANES) * LANES

        @pl.loop(0, n_groups)
        def _(g):
            r0 = g * LANES
            for j in range(LANES):
                u_lo = urows_v[r0 + j, pl.ds(0, LANES)]
                u_hi = urows_v[r0 + j, pl.ds(LANES, LANES)]
                v_lo = irows_v[r0 + j, pl.ds(0, LANES)]
                v_hi = irows_v[r0 + j, pl.ds(LANES, LANES)]
                q_v[pl.ds(j * LANES, LANES)] = (
                    u_lo * v_lo * w_lo + u_hi * v_hi * w_hi)
            acc = bvec
            for d in range(LANES):
                acc = acc + plsc.load_gather(q_v, [lane_off + d])
            scores_v[pl.ds(r0, LANES)] = acc

        pltpu.sync_copy(scores_v, out_hbm.at[pl.ds(base, b_per_w)])

    return k(uidx, iidx, user_table, item_table, w_flat, bias16)


def kernel(user_indices, item_indices, user_table, item_table, affine_W,
           affine_b):
    uidx = user_indices.astype(jnp.int32)
    iidx = item_indices.astype(jnp.int32)
    w_flat = affine_W.reshape(LATENT).astype(jnp.float32)
    bias16 = jnp.broadcast_to(affine_b.astype(jnp.float32), (LANES,))
    scores = _scores_sc(uidx, iidx, user_table, item_table, w_flat, bias16)
    return scores.reshape(user_indices.shape[0], 1)


# speedup vs baseline: 3.8284x; 3.8284x over previous
import dataclasses
import functools

import jax
import jax.numpy as jnp
from jax import lax
from jax.experimental import pallas as pl
from jax.experimental.pallas import tpu as pltpu
from jax.experimental.pallas import tpu_sc as plsc

LATENT = 32
LANES = 16
NUM_CORES = 2
NUM_SUBCORES = 16
NUM_WORKERS = NUM_CORES * NUM_SUBCORES
SUB = 4
NSUB = 4


def _scores_sc(uidx, iidx, ut_t, it_t, w_flat, bias16):
    batch = uidx.shape[0]
    b_per_w = batch // NUM_WORKERS
    n_groups = b_per_w // LANES

    mesh = plsc.VectorSubcoreMesh(core_axis_name="c", subcore_axis_name="s")
    cparams = pltpu.CompilerParams()
    if "needs_layout_passes" in pltpu.CompilerParams.__dataclass_fields__:
        cparams = dataclasses.replace(cparams, needs_layout_passes=False)

    @functools.partial(
        pl.kernel,
        mesh=mesh,
        compiler_params=cparams,
        out_type=jax.ShapeDtypeStruct((batch,), jnp.float32),
        scratch_types=[
            pltpu.VMEM((b_per_w,), jnp.int32),
            pltpu.VMEM((b_per_w,), jnp.int32),
            pltpu.VMEM((2, SUB, LATENT, 128), jnp.float32),
            pltpu.VMEM((2, SUB, LATENT, 128), jnp.float32),
            pltpu.VMEM((b_per_w * LANES // SUB,), jnp.float32),
            pltpu.VMEM((b_per_w,), jnp.float32),
            pltpu.VMEM((LATENT,), jnp.float32),
            pltpu.VMEM((LANES,), jnp.float32),
            pltpu.SemaphoreType.DMA,
            pltpu.SemaphoreType.DMA,
            pltpu.SemaphoreType.DMA,
            pltpu.SemaphoreType.DMA,
        ],
    )
    def k(uidx_hbm, iidx_hbm, ut_hbm, it_hbm, w_hbm, b_hbm, out_hbm,
          uidx_v, iidx_v, ublk, iblk, acc_v, scores_v, w_v, b_v,
          semu0, semu1, semi0, semi1):
        wid = lax.axis_index("s") * NUM_CORES + lax.axis_index("c")
        base = wid * b_per_w

        pltpu.sync_copy(uidx_hbm.at[pl.ds(base, b_per_w)], uidx_v)
        pltpu.sync_copy(iidx_hbm.at[pl.ds(base, b_per_w)], iidx_v)
        pltpu.sync_copy(w_hbm, w_v)
        pltpu.sync_copy(b_hbm, b_v)

        semu = (semu0, semu1)
        semi = (semi0, semi1)
        iota = lax.iota(jnp.int32, LANES)
        jsel = iota // SUB
        ddsel = iota % SUB
        bvec = b_v[...]
        wseg = [plsc.load_gather(w_v, [ddsel + dg * SUB])
                for dg in range(LATENT // SUB)]

        def issue(gg, scn, pp):
            uvec = uidx_v[pl.ds(gg * LANES, LANES)]
            ivec = iidx_v[pl.ds(gg * LANES, LANES)]
            uoffs = (uvec >> 7) << 7
            ioffs = (ivec >> 7) << 7
            for j in range(SUB):
                lane = scn * SUB + j
                uo = pl.multiple_of(uoffs[lane], 128)
                io = pl.multiple_of(ioffs[lane], 128)
                pltpu.async_copy(ut_hbm.at[:, pl.ds(uo, 128)],
                                 ublk.at[pp, j], semu[pp])
                pltpu.async_copy(it_hbm.at[:, pl.ds(io, 128)],
                                 iblk.at[pp, j], semi[pp])

        def drain(pp):
            for j in range(SUB):
                pltpu.make_async_copy(ut_hbm.at[:, pl.ds(0, 128)],
                                      ublk.at[pp, j], semu[pp]).wait()
                pltpu.make_async_copy(it_hbm.at[:, pl.ds(0, 128)],
                                      iblk.at[pp, j], semi[pp]).wait()

        issue(0, 0, 0)

        @pl.loop(0, n_groups)
        def _(g):
            for sc in range(NSUB):
                par = sc % 2
                nxt = (sc + 1) % 2
                scn = (sc + 1) % NSUB
                if scn == 0:
                    @pl.when(g < n_groups - 1)
                    def _():
                        issue(g + 1, 0, nxt)
                else:
                    issue(g, scn, nxt)
                drain(par)
                cpos = g * LANES + sc * SUB + jsel
                lmu = plsc.load_gather(uidx_v, [cpos]) & 127
                lmi = plsc.load_gather(iidx_v, [cpos]) & 127
                pvec = jnp.full((LANES,), par, jnp.int32)
                acc = jnp.zeros((LANES,), jnp.float32)
                for dg in range(LATENT // SUB):
                    dvec = ddsel + dg * SUB
                    gu = plsc.load_gather(ublk, [pvec, jsel, dvec, lmu])
                    gv = plsc.load_gather(iblk, [pvec, jsel, dvec, lmi])
                    acc = acc + gu * gv * wseg[dg]
                acc_v[pl.ds((g * NSUB + sc) * LANES, LANES)] = acc

        qsel = (iota // SUB) * LANES + (iota % SUB) * SUB

        @pl.loop(0, n_groups)
        def _(og):
            idx0 = og * (NSUB * LANES) + qsel
            s = bvec
            for dd in range(SUB):
                s = s + plsc.load_gather(acc_v, [idx0 + dd])
            scores_v[pl.ds(og * LANES, LANES)] = s

        pltpu.sync_copy(scores_v, out_hbm.at[pl.ds(base, b_per_w)])

    return k(uidx, iidx, ut_t, it_t, w_flat, bias16)


def kernel(user_indices, item_indices, user_table, item_table, affine_W,
           affine_b):
    uidx = user_indices.astype(jnp.int32)
    iidx = item_indices.astype(jnp.int32)
    w_flat = affine_W.reshape(LATENT).astype(jnp.float32)
    bias16 = jnp.broadcast_to(affine_b.astype(jnp.float32), (LANES,))
    scores = _scores_sc(uidx, iidx, user_table.T, item_table.T, w_flat,
                        bias16)
    return scores.reshape(user_indices.shape[0], 1)


# speedup vs baseline: 4.1899x; 1.0944x over previous
import dataclasses
import functools

import jax
import jax.numpy as jnp
from jax import lax
from jax.experimental import pallas as pl
from jax.experimental.pallas import tpu as pltpu
from jax.experimental.pallas import tpu_sc as plsc

LATENT = 32
LANES = 16
NUM_CORES = 2
NUM_SUBCORES = 16
NUM_WORKERS = NUM_CORES * NUM_SUBCORES
SUB = 2
NSUB = LANES // SUB
RING = 4
DIST = 3
DCLS = LANES // SUB


def _scores_sc(uidx, iidx, ut_t, it_t, w_flat, bias16):
    batch = uidx.shape[0]
    b_per_w = batch // NUM_WORKERS
    n_groups = b_per_w // LANES

    mesh = plsc.VectorSubcoreMesh(core_axis_name="c", subcore_axis_name="s")
    cparams = pltpu.CompilerParams()
    if "needs_layout_passes" in pltpu.CompilerParams.__dataclass_fields__:
        cparams = dataclasses.replace(cparams, needs_layout_passes=False)

    @functools.partial(
        pl.kernel,
        mesh=mesh,
        compiler_params=cparams,
        out_type=jax.ShapeDtypeStruct((batch,), jnp.float32),
        scratch_types=[
            pltpu.VMEM((b_per_w,), jnp.int32),
            pltpu.VMEM((b_per_w,), jnp.int32),
            pltpu.VMEM((RING, SUB, LATENT, 128), jnp.float32),
            pltpu.VMEM((RING, SUB, LATENT, 128), jnp.float32),
            pltpu.VMEM((b_per_w * LANES // SUB,), jnp.float32),
            pltpu.VMEM((b_per_w,), jnp.float32),
            pltpu.VMEM((LATENT,), jnp.float32),
            pltpu.VMEM((LANES,), jnp.float32),
        ] + [pltpu.SemaphoreType.DMA] * (2 * RING),
    )
    def k(uidx_hbm, iidx_hbm, ut_hbm, it_hbm, w_hbm, b_hbm, out_hbm,
          uidx_v, iidx_v, ublk, iblk, acc_v, scores_v, w_v, b_v, *sems):
        wid = lax.axis_index("s") * NUM_CORES + lax.axis_index("c")
        base = wid * b_per_w

        pltpu.sync_copy(uidx_hbm.at[pl.ds(base, b_per_w)], uidx_v)
        pltpu.sync_copy(iidx_hbm.at[pl.ds(base, b_per_w)], iidx_v)
        pltpu.sync_copy(w_hbm, w_v)
        pltpu.sync_copy(b_hbm, b_v)

        semu = sems[:RING]
        semi = sems[RING:]
        iota = lax.iota(jnp.int32, LANES)
        jsel = iota // DCLS
        ddsel = iota % DCLS
        bvec = b_v[...]
        wseg = [plsc.load_gather(w_v, [ddsel + dg * DCLS])
                for dg in range(LATENT // DCLS)]

        def issue(gg, scn, pp):
            uvec = uidx_v[pl.ds(gg * LANES, LANES)]
            ivec = iidx_v[pl.ds(gg * LANES, LANES)]
            uoffs = (uvec >> 7) << 7
            ioffs = (ivec >> 7) << 7
            for j in range(SUB):
                lane = scn * SUB + j
                uo = pl.multiple_of(uoffs[lane], 128)
                io = pl.multiple_of(ioffs[lane], 128)
                pltpu.async_copy(ut_hbm.at[:, pl.ds(uo, 128)],
                                 ublk.at[pp, j], semu[pp])
                pltpu.async_copy(it_hbm.at[:, pl.ds(io, 128)],
                                 iblk.at[pp, j], semi[pp])

        def drain(pp):
            for j in range(SUB):
                pltpu.make_async_copy(ut_hbm.at[:, pl.ds(0, 128)],
                                      ublk.at[pp, j], semu[pp]).wait()
                pltpu.make_async_copy(it_hbm.at[:, pl.ds(0, 128)],
                                      iblk.at[pp, j], semi[pp]).wait()

        for n in range(DIST):
            issue(n // NSUB, n % NSUB, n % RING)

        @pl.loop(0, n_groups)
        def _(g):
            for sc in range(NSUB):
                par = sc % RING
                tgt_sc = (sc + DIST) % NSUB
                tgt_pp = (sc + DIST) % RING
                if (sc + DIST) // NSUB:
                    @pl.when(g < n_groups - 1)
                    def _():
                        issue(g + 1, tgt_sc, tgt_pp)
                else:
                    issue(g, tgt_sc, tgt_pp)
                drain(par)
                cpos = g * LANES + sc * SUB + jsel
                lmu = plsc.load_gather(uidx_v, [cpos]) & 127
                lmi = plsc.load_gather(iidx_v, [cpos]) & 127
                pvec = jnp.full((LANES,), par, jnp.int32)
                acc = jnp.zeros((LANES,), jnp.float32)
                for dg in range(LATENT // DCLS):
                    dvec = ddsel + dg * DCLS
                    gu = plsc.load_gather(ublk, [pvec, jsel, dvec, lmu])
                    gv = plsc.load_gather(iblk, [pvec, jsel, dvec, lmi])
                    acc = acc + gu * gv * wseg[dg]
                acc_v[pl.ds((g * NSUB + sc) * LANES, LANES)] = acc

        qsel = (iota // SUB) * LANES + (iota % SUB) * DCLS

        @pl.loop(0, n_groups)
        def _(og):
            idx0 = og * (NSUB * LANES) + qsel
            s = bvec
            for dd in range(DCLS):
                s = s + plsc.load_gather(acc_v, [idx0 + dd])
            scores_v[pl.ds(og * LANES, LANES)] = s

        pltpu.sync_copy(scores_v, out_hbm.at[pl.ds(base, b_per_w)])

    return k(uidx, iidx, ut_t, it_t, w_flat, bias16)


def kernel(user_indices, item_indices, user_table, item_table, affine_W,
           affine_b):
    uidx = user_indices.astype(jnp.int32)
    iidx = item_indices.astype(jnp.int32)
    w_flat = affine_W.reshape(LATENT).astype(jnp.float32)
    bias16 = jnp.broadcast_to(affine_b.astype(jnp.float32), (LANES,))
    scores = _scores_sc(uidx, iidx, user_table.T, item_table.T, w_flat,
                        bias16)
    return scores.reshape(user_indices.shape[0], 1)
